# Initial kernel scaffold; baseline (speedup 1.0000x reference)
import jax
import jax.numpy as jnp
from jax.experimental import pallas as pl


def kernel(x, edge_index, edge_attr, restriction_maps):
    raise NotImplementedError("write your pallas kernel here")


# speedup vs baseline: 19.6738x; 19.6738x over previous
import jax
import jax.numpy as jnp
from jax.experimental import pallas as pl

N_NODES = 100000
STALK = 8
NFEAT = 4
NATTR = 16
EB = 4096


def _compute_block(attrT_ref, xuT_ref, xvT_ref, bpj_ref, bpi_ref, out_ref):
    a_t = attrT_ref[...]
    gj = jnp.dot(bpj_ref[...], a_t, preferred_element_type=jnp.float32,
                 precision=jax.lax.Precision.HIGHEST)
    gi = jnp.dot(bpi_ref[...], a_t, preferred_element_type=jnp.float32,
                 precision=jax.lax.Precision.HIGHEST)
    xu_t = xuT_ref[...]
    xv_t = xvT_ref[...]
    cu = [None] * NFEAT
    cv = [None] * NFEAT
    for f in range(NFEAT):
        dxf = gj[0:8, :] * xu_t[f:f + 1, :]
        for j in range(1, STALK):
            dxf = dxf + gj[8 * j:8 * j + 8, :] * xu_t[4 * j + f:4 * j + f + 1, :]
        for j in range(STALK):
            dxf = dxf + gj[64 + 8 * j:64 + 8 * j + 8, :] * xv_t[4 * j + f:4 * j + f + 1, :]
        cuf = gi[0:8, :] * dxf[0:1, :]
        cvf = gi[8:16, :] * dxf[0:1, :]
        for i in range(1, STALK):
            cuf = cuf + gi[16 * i:16 * i + 8, :] * dxf[i:i + 1, :]
            cvf = cvf + gi[16 * i + 8:16 * i + 16, :] * dxf[i:i + 1, :]
        cu[f] = cuf
        cv[f] = cvf
    out_ref[...] = jnp.concatenate(cu + cv, axis=0)


def kernel(x, edge_index, edge_attr, restriction_maps):
    E = edge_attr.shape[0]
    EP = ((E + EB - 1) // EB) * EB
    nb = EP // EB

    attr_p = jnp.pad(edge_attr, ((0, EP - E), (0, 0)))
    attrT = attr_p.T
    up = jnp.pad(edge_index[0], (0, EP - E))
    vp = jnp.pad(edge_index[1], (0, EP - E))

    xflat = x.reshape(N_NODES, STALK * NFEAT)
    xuT = jnp.take(xflat, up, axis=0).T
    xvT = jnp.take(xflat, vp, axis=0).T

    rm0 = restriction_maps[:, 0]
    rm1 = restriction_maps[:, 1]
    bpj = jnp.concatenate([
        jnp.transpose(rm0, (2, 1, 0)).reshape(64, NATTR),
        -jnp.transpose(rm1, (2, 1, 0)).reshape(64, NATTR),
    ], axis=0)
    bpi = jnp.concatenate([
        jnp.transpose(rm0, (1, 2, 0)),
        -jnp.transpose(rm1, (1, 2, 0)),
    ], axis=1).reshape(128, NATTR)

    contribT = pl.pallas_call(
        _compute_block,
        grid=(nb,),
        in_specs=[
            pl.BlockSpec((NATTR, EB), lambda b: (0, b)),
            pl.BlockSpec((2 * NATTR, EB), lambda b: (0, b)),
            pl.BlockSpec((2 * NATTR, EB), lambda b: (0, b)),
            pl.BlockSpec((128, NATTR), lambda b: (0, 0)),
            pl.BlockSpec((128, NATTR), lambda b: (0, 0)),
        ],
        out_specs=pl.BlockSpec((64, EB), lambda b: (0, b)),
        out_shape=jax.ShapeDtypeStruct((64, EP), jnp.float32),
    )(attrT, xuT, xvT, bpj, bpi)

    contrib = contribT.T
    outflat = jnp.zeros((N_NODES, 32), jnp.float32)
    outflat = outflat.at[up].add(contrib[:, :32])
    outflat = outflat.at[vp].add(contrib[:, 32:])
    out = outflat.reshape(N_NODES, NFEAT, STALK).transpose(0, 2, 1)
    return out


# speedup vs baseline: 19.8544x; 1.0092x over previous
import jax
import jax.numpy as jnp
from jax import lax
from jax.experimental import pallas as pl
from jax.experimental.pallas import tpu as pltpu
from jax.experimental.pallas import tpu_sc as plsc

N_NODES = 100000
STALK = 8
NFEAT = 4
NATTR = 16
EB = 4096
CH = 1280
K = CH // 128
NR = 100096
RPT = NR // 16


def _compute_block(attrT_ref, xuT_ref, xvT_ref, bpj_ref, bpi_ref, out_ref):
    a_t = attrT_ref[...]
    gj = jnp.dot(bpj_ref[...], a_t, preferred_element_type=jnp.float32,
                 precision=jax.lax.Precision.HIGHEST)
    gi = jnp.dot(bpi_ref[...], a_t, preferred_element_type=jnp.float32,
                 precision=jax.lax.Precision.HIGHEST)
    xu_t = xuT_ref[...]
    xv_t = xvT_ref[...]
    cu = [None] * NFEAT
    cv = [None] * NFEAT
    for f in range(NFEAT):
        dxf = gj[0:8, :] * xu_t[f:f + 1, :]
        for j in range(1, STALK):
            dxf = dxf + gj[8 * j:8 * j + 8, :] * xu_t[4 * j + f:4 * j + f + 1, :]
        for j in range(STALK):
            dxf = dxf + gj[64 + 8 * j:64 + 8 * j + 8, :] * xv_t[4 * j + f:4 * j + f + 1, :]
        cuf = gi[0:8, :] * dxf[0:1, :]
        cvf = gi[8:16, :] * dxf[0:1, :]
        for i in range(1, STALK):
            cuf = cuf + gi[16 * i:16 * i + 8, :] * dxf[i:i + 1, :]
            cvf = cvf + gi[16 * i + 8:16 * i + 16, :] * dxf[i:i + 1, :]
        cu[f] = cuf
        cv[f] = cvf
    out_ref[...] = jnp.concatenate(cu + cv, axis=0)


def _scatter_body(contrib_ref, up_ref, vp_ref, zeros_ref, out_ref,
                  idx_v, data_v, acc, sem):
    c = lax.axis_index("c")
    s = lax.axis_index("s")
    ep = up_ref.shape[0] * 128
    rows_per_tile = ep // 16
    nchunk = rows_per_tile // CH
    pltpu.sync_copy(zeros_ref, acc.at[pl.ds(pl.multiple_of(s * RPT, 16), RPT)])
    plsc.subcore_barrier()

    def do_piece(idx_hbm, piece):
        def chunk(it, carry):
            base = pl.multiple_of(s * rows_per_tile + it * CH, 128)
            rowb = pl.multiple_of(base // 128, 2)
            pltpu.sync_copy(idx_hbm.at[pl.ds(rowb, K), :], idx_v)
            pltpu.sync_copy(contrib_ref.at[piece, pl.ds(base, CH), :], data_v)
            for j in range(K):
                pltpu.sync_copy(data_v.at[pl.ds(j * 128, 128), :],
                                acc.at[idx_v.at[j]], add=True)
            return carry
        lax.fori_loop(0, nchunk, chunk, 0)

    do_piece(up_ref, c)
    do_piece(vp_ref, 2 + c)
    plsc.subcore_barrier()
    @pl.when(s < 15)
    def _():
        off = pl.multiple_of(s * RPT, 16)
        pltpu.sync_copy(acc.at[pl.ds(off, RPT)],
                        out_ref.at[c, pl.ds(off, RPT), :])
    @pl.when(s == 15)
    def _():
        pltpu.sync_copy(acc.at[pl.ds(15 * RPT, N_NODES - 15 * RPT)],
                        out_ref.at[c, pl.ds(15 * RPT, N_NODES - 15 * RPT), :])


def kernel(x, edge_index, edge_attr, restriction_maps):
    E = edge_attr.shape[0]
    EP = ((E + EB - 1) // EB) * EB
    nb = EP // EB

    attr_p = jnp.pad(edge_attr, ((0, EP - E), (0, 0)))
    attrT = attr_p.T
    dummy = N_NODES + (jnp.arange(EP - E, dtype=jnp.int32) % (NR - N_NODES))
    up = jnp.concatenate([edge_index[0], dummy])
    vp = jnp.concatenate([edge_index[1], dummy])
    up_g = jnp.where(up >= N_NODES, 0, up)
    vp_g = jnp.where(vp >= N_NODES, 0, vp)

    xflat = x.reshape(N_NODES, STALK * NFEAT)
    xuT = jnp.take(xflat, up_g, axis=0).T
    xvT = jnp.take(xflat, vp_g, axis=0).T

    rm0 = restriction_maps[:, 0]
    rm1 = restriction_maps[:, 1]
    bpj = jnp.concatenate([
        jnp.transpose(rm0, (2, 1, 0)).reshape(64, NATTR),
        -jnp.transpose(rm1, (2, 1, 0)).reshape(64, NATTR),
    ], axis=0)
    bpi = jnp.concatenate([
        jnp.transpose(rm0, (1, 2, 0)),
        -jnp.transpose(rm1, (1, 2, 0)),
    ], axis=1).reshape(128, NATTR)

    contribT = pl.pallas_call(
        _compute_block,
        grid=(nb,),
        in_specs=[
            pl.BlockSpec((NATTR, EB), lambda b: (0, b)),
            pl.BlockSpec((2 * NATTR, EB), lambda b: (0, b)),
            pl.BlockSpec((2 * NATTR, EB), lambda b: (0, b)),
            pl.BlockSpec((128, NATTR), lambda b: (0, 0)),
            pl.BlockSpec((128, NATTR), lambda b: (0, 0)),
        ],
        out_specs=pl.BlockSpec((64, EB), lambda b: (0, b)),
        out_shape=jax.ShapeDtypeStruct((64, EP), jnp.float32),
    )(attrT, xuT, xvT, bpj, bpi)

    contrib = jnp.transpose(contribT.reshape(4, 16, EP), (0, 2, 1))
    if True:
        cflat = contribT.T
        outflat = jnp.zeros((N_NODES, 32), jnp.float32)
        outflat = outflat.at[up_g].add(cflat[:, :32])
        outflat = outflat.at[vp_g].add(cflat[:, 32:])
        return outflat.reshape(N_NODES, NFEAT, STALK).transpose(0, 2, 1)

    mesh = plsc.VectorSubcoreMesh(core_axis_name="c", subcore_axis_name="s",
                                  num_cores=2, num_subcores=16)
    out2 = pl.kernel(
        _scatter_body,
        out_type=jax.ShapeDtypeStruct((2, N_NODES, 16), jnp.float32),
        mesh=mesh,
        scratch_types=[
            pltpu.VMEM((K, 128), jnp.int32),
            pltpu.VMEM((CH, 16), jnp.float32),
            pltpu.VMEM_SHARED((NR, 16), jnp.float32),
            pltpu.SemaphoreType.DMA,
        ],
        compiler_params=pltpu.CompilerParams(use_tc_tiling_on_sc=False),
    )(contrib, up.reshape(EP // 128, 128), vp.reshape(EP // 128, 128),
      jnp.zeros((RPT, 16), jnp.float32))

    outflat = jnp.concatenate([out2[0], out2[1]], axis=1)
    out = outflat.reshape(N_NODES, NFEAT, STALK).transpose(0, 2, 1)
    return out


# speedup vs baseline: 54.5842x; 2.7492x over previous
import jax
import jax.numpy as jnp
from jax import lax
from jax.experimental import pallas as pl
from jax.experimental.pallas import tpu as pltpu
from jax.experimental.pallas import tpu_sc as plsc

N_NODES = 100000
STALK = 8
NFEAT = 4
NATTR = 16
EB = 4096
CH = 1280
K = CH // 128
NR = 100096
RPT = NR // 16


def _compute_block(attrT_ref, xuT_ref, xvT_ref, bpj_ref, bpi_ref, out_ref):
    a_t = attrT_ref[...]
    gj = jnp.dot(bpj_ref[...], a_t, preferred_element_type=jnp.float32,
                 precision=jax.lax.Precision.HIGHEST)
    gi = jnp.dot(bpi_ref[...], a_t, preferred_element_type=jnp.float32,
                 precision=jax.lax.Precision.HIGHEST)
    xu_t = xuT_ref[...]
    xv_t = xvT_ref[...]
    cu = [None] * NFEAT
    cv = [None] * NFEAT
    for f in range(NFEAT):
        dxf = gj[0:8, :] * xu_t[f:f + 1, :]
        for j in range(1, STALK):
            dxf = dxf + gj[8 * j:8 * j + 8, :] * xu_t[4 * j + f:4 * j + f + 1, :]
        for j in range(STALK):
            dxf = dxf + gj[64 + 8 * j:64 + 8 * j + 8, :] * xv_t[4 * j + f:4 * j + f + 1, :]
        cuf = gi[0:8, :] * dxf[0:1, :]
        cvf = gi[8:16, :] * dxf[0:1, :]
        for i in range(1, STALK):
            cuf = cuf + gi[16 * i:16 * i + 8, :] * dxf[i:i + 1, :]
            cvf = cvf + gi[16 * i + 8:16 * i + 16, :] * dxf[i:i + 1, :]
        cu[f] = cuf
        cv[f] = cvf
    pieces = [
        jnp.concatenate([cu[0], cu[1]], axis=0),
        jnp.concatenate([cu[2], cu[3]], axis=0),
        jnp.concatenate([cv[0], cv[1]], axis=0),
        jnp.concatenate([cv[2], cv[3]], axis=0),
    ]
    out_ref[...] = jnp.stack([p.T for p in pieces], axis=0)


def _gather_body(xflat_ref, up_ref, vp_ref, out_ref, idx_v, rows_v, sem):
    c = lax.axis_index("c")
    s = lax.axis_index("s")
    w = s * 2 + c
    ep = up_ref.shape[0] * 128
    rows_per_tile = ep // 32
    nchunk = rows_per_tile // CH

    def do_side(idx_hbm, side):
        def chunk(it, carry):
            base = pl.multiple_of(w * rows_per_tile + it * CH, 128)
            rowb = pl.multiple_of(base // 128, 2)
            pltpu.sync_copy(idx_hbm.at[pl.ds(rowb, K), :], idx_v)
            cps = [
                pltpu.async_copy(xflat_ref.at[idx_v.at[j]],
                                 rows_v.at[pl.ds(j * 128, 128), :], sem)
                for j in range(K)
            ]
            for cp in cps:
                cp.wait()
            pltpu.sync_copy(rows_v, out_ref.at[side, pl.ds(base, CH), :])
            return carry
        lax.fori_loop(0, nchunk, chunk, 0)

    do_side(up_ref, 0)
    do_side(vp_ref, 1)


def _scatter_body(contrib_ref, up_ref, vp_ref, zeros_ref, out_ref,
                  idx_v, data_v, acc, sem):
    c = lax.axis_index("c")
    s = lax.axis_index("s")
    ep = up_ref.shape[0] * 128
    rows_per_tile = ep // 16
    nchunk = rows_per_tile // CH
    pltpu.sync_copy(zeros_ref, acc.at[pl.ds(pl.multiple_of(s * RPT, 16), RPT)])
    plsc.subcore_barrier()

    def do_piece(idx_hbm, piece):
        def chunk(it, carry):
            base = pl.multiple_of(s * rows_per_tile + it * CH, 128)
            rowb = pl.multiple_of(base // 128, 2)
            pltpu.sync_copy(idx_hbm.at[pl.ds(rowb, K), :], idx_v)
            pltpu.sync_copy(contrib_ref.at[piece, pl.ds(base, CH), :], data_v)
            for j in range(K):
                pltpu.sync_copy(data_v.at[pl.ds(j * 128, 128), :],
                                acc.at[idx_v.at[j]], add=True)
            return carry
        lax.fori_loop(0, nchunk, chunk, 0)

    do_piece(up_ref, c)
    do_piece(vp_ref, 2 + c)
    plsc.subcore_barrier()
    @pl.when(s < 15)
    def _():
        off = pl.multiple_of(s * RPT, 16)
        pltpu.sync_copy(acc.at[pl.ds(off, RPT)],
                        out_ref.at[c, pl.ds(off, RPT), :])
    @pl.when(s == 15)
    def _():
        pltpu.sync_copy(acc.at[pl.ds(15 * RPT, N_NODES - 15 * RPT)],
                        out_ref.at[c, pl.ds(15 * RPT, N_NODES - 15 * RPT), :])


def kernel(x, edge_index, edge_attr, restriction_maps):
    E = edge_attr.shape[0]
    EP = ((E + EB - 1) // EB) * EB
    nb = EP // EB

    attr_p = jnp.pad(edge_attr, ((0, EP - E), (0, 0)))
    attrT = attr_p.T
    dummy = jnp.arange(EP - E, dtype=jnp.int32) * 997 % N_NODES
    up = jnp.concatenate([edge_index[0], dummy])
    vp = jnp.concatenate([edge_index[1], dummy])
    up_g = jnp.where(up >= N_NODES, 0, up)
    vp_g = jnp.where(vp >= N_NODES, 0, vp)

    xflat = x.reshape(N_NODES, STALK * NFEAT)
    mesh_g = plsc.VectorSubcoreMesh(core_axis_name="c", subcore_axis_name="s",
                                    num_cores=2, num_subcores=16)
    xg = pl.kernel(
        _gather_body,
        out_type=jax.ShapeDtypeStruct((2, EP, 32), jnp.float32),
        mesh=mesh_g,
        scratch_types=[
            pltpu.VMEM((K, 128), jnp.int32),
            pltpu.VMEM((CH, 32), jnp.float32),
            pltpu.SemaphoreType.DMA,
        ],
        compiler_params=pltpu.CompilerParams(use_tc_tiling_on_sc=False),
    )(xflat, up_g.reshape(EP // 128, 128), vp_g.reshape(EP // 128, 128))
    xuT = xg[0].T
    xvT = xg[1].T

    rm0 = restriction_maps[:, 0]
    rm1 = restriction_maps[:, 1]
    bpj = jnp.concatenate([
        jnp.transpose(rm0, (2, 1, 0)).reshape(64, NATTR),
        -jnp.transpose(rm1, (2, 1, 0)).reshape(64, NATTR),
    ], axis=0)
    bpi = jnp.concatenate([
        jnp.transpose(rm0, (1, 2, 0)),
        -jnp.transpose(rm1, (1, 2, 0)),
    ], axis=1).reshape(128, NATTR)

    contribT = pl.pallas_call(
        _compute_block,
        grid=(nb,),
        in_specs=[
            pl.BlockSpec((NATTR, EB), lambda b: (0, b)),
            pl.BlockSpec((2 * NATTR, EB), lambda b: (0, b)),
            pl.BlockSpec((2 * NATTR, EB), lambda b: (0, b)),
            pl.BlockSpec((128, NATTR), lambda b: (0, 0)),
            pl.BlockSpec((128, NATTR), lambda b: (0, 0)),
        ],
        out_specs=pl.BlockSpec((4, EB, 16), lambda b: (0, b, 0)),
        out_shape=jax.ShapeDtypeStruct((4, EP, 16), jnp.float32),
    )(attrT, xuT, xvT, bpj, bpi)
    contrib = contribT

    contrib, up_sc, vp_sc = lax.optimization_barrier(
        (contrib, up.reshape(EP // 128, 128), vp.reshape(EP // 128, 128)))

    mesh = plsc.VectorSubcoreMesh(core_axis_name="c", subcore_axis_name="s",
                                  num_cores=2, num_subcores=16)
    out2 = pl.kernel(
        _scatter_body,
        out_type=jax.ShapeDtypeStruct((2, N_NODES, 16), jnp.float32),
        mesh=mesh,
        scratch_types=[
            pltpu.VMEM((K, 128), jnp.int32),
            pltpu.VMEM((CH, 16), jnp.float32),
            pltpu.VMEM_SHARED((NR, 16), jnp.float32),
            pltpu.SemaphoreType.DMA,
        ],
        compiler_params=pltpu.CompilerParams(use_tc_tiling_on_sc=False),
    )(contrib, up_sc, vp_sc,
      jnp.zeros((RPT, 16), jnp.float32))

    outflat = jnp.concatenate([out2[0], out2[1]], axis=1)
    out = outflat.reshape(N_NODES, NFEAT, STALK).transpose(0, 2, 1)
    return out
